# baseline (speedup 1.0000x reference)
import jax
import jax.numpy as jnp
from jax.experimental import pallas as pl
from jax.experimental.pallas import tpu as pltpu
from jax.experimental.pallas import tpu_sc as plsc

_B, _C, _H, _W, _E = 2, 192, 224, 224, 8
_HW = _H * _W
_NB1 = 3584
_NB2 = 3584
_CTC = 160
_CSC = _C - _CTC
_NROWS = _B * _CSC
_NWORK = 32
_RPW = _NROWS // _NWORK
_CHUNK = 6272
_NCH = _HW // _CHUNK


def _sc_partial_sums(x2):
    mesh = plsc.VectorSubcoreMesh(core_axis_name="core",
                                  subcore_axis_name="subcore")

    @pl.kernel(out_type=jax.ShapeDtypeStruct((_B, _CSC, 16), jnp.float32),
               mesh=mesh,
               scratch_types=[pltpu.VMEM((2, _CHUNK), jnp.float32),
                              pltpu.VMEM((4, 16), jnp.float32),
                              pltpu.SemaphoreType.DMA((2,)),
                              pltpu.SemaphoreType.DMA])
    def sc_reduce(x_hbm, o_hbm, buf_ref, acc_ref, sems, osem):
        core = jax.lax.axis_index("core")
        sub = jax.lax.axis_index("subcore")
        base = (core * 16 + sub) * _RPW

        def accumulate(slot):
            @pl.loop(0, _CHUNK // 16, step=4)
            def _(t):
                for u in range(4):
                    acc_ref.at[u][...] += buf_ref.at[slot][
                        pl.ds((t + u) * 16, 16)]

        @pl.loop(0, _RPW)
        def _(k):
            rid = base + k
            b = rid // _CSC
            cc = rid - b * _CSC
            flat = b * _C + _CTC + cc
            for u in range(4):
                acc_ref.at[u][...] = jnp.zeros((16,), jnp.float32)
            pltpu.async_copy(x_hbm.at[flat, pl.ds(0, _CHUNK)],
                             buf_ref.at[0], sems.at[0])

            @pl.loop(0, _NCH // 2)
            def _(q):
                pltpu.async_copy(
                    x_hbm.at[flat, pl.ds((2 * q + 1) * _CHUNK, _CHUNK)],
                    buf_ref.at[1], sems.at[1])
                pltpu.make_async_copy(
                    x_hbm.at[flat, pl.ds(2 * q * _CHUNK, _CHUNK)],
                    buf_ref.at[0], sems.at[0]).wait()
                accumulate(0)

                @pl.when(q + 1 < _NCH // 2)
                def _():
                    pltpu.async_copy(
                        x_hbm.at[flat, pl.ds((2 * q + 2) * _CHUNK, _CHUNK)],
                        buf_ref.at[0], sems.at[0])

                pltpu.make_async_copy(
                    x_hbm.at[flat, pl.ds((2 * q + 1) * _CHUNK, _CHUNK)],
                    buf_ref.at[1], sems.at[1]).wait()
                accumulate(1)

            acc_ref.at[0][...] += acc_ref.at[1][...]
            acc_ref.at[2][...] += acc_ref.at[3][...]
            acc_ref.at[0][...] += acc_ref.at[2][...]
            pltpu.async_copy(acc_ref.at[0], o_hbm.at[b, cc], osem).wait()

    return sc_reduce(x2)


def _tc_reduce_body(x_ref, sums_ref, acc_ref):
    h = pl.program_id(0)

    @pl.when(h == 0)
    def _():
        acc_ref[...] = jnp.zeros_like(acc_ref)

    acc_ref[...] += jnp.sum(x_ref[...], axis=2)

    @pl.when(h == pl.num_programs(0) - 1)
    def _():
        sums_ref[...] = acc_ref[...]


def _combine_body(tcs_ref, scp_ref, wctl_ref, bctl_ref, dec_ref):
    w_tc = wctl_ref[:, :_CTC]
    w_sc = wctl_ref[:, _CTC:]
    ctl = jax.lax.dot_general(
        tcs_ref[...], w_tc, (((1,), (1,)), ((), ())),
        preferred_element_type=jnp.float32)
    sc_sums = jnp.sum(scp_ref[...], axis=2)
    ctl += jax.lax.dot_general(
        sc_sums, w_sc, (((1,), (1,)), ((), ())),
        preferred_element_type=jnp.float32)
    ctl = ctl * (1.0 / _HW) + bctl_ref[...]
    mx = jnp.max(ctl, axis=1, keepdims=True)
    idx = jax.lax.broadcasted_iota(jnp.int32, (_B, _E), 1)
    dec_ref[0, :] = jnp.min(jnp.where(ctl == mx, idx, _E), axis=1)


def _expert_body(dec_ref, x_ref, w0_ref, w1_ref, b0_ref, b1_ref, o_ref):
    dims = (((1,), (0,)), ((), ()))
    for i, (w_ref, b_ref) in enumerate(((w0_ref, b0_ref), (w1_ref, b1_ref))):
        w = w_ref[0]
        bias = b_ref[0]
        for b in range(_B):
            y = jax.lax.dot_general(w, x_ref[b], dims,
                                    preferred_element_type=jnp.float32)
            o_ref[i * _B + b] = y + bias


def kernel(x, W_ctl, b_ctl, W_comp, b_comp):
    x3 = x.reshape(_B, _C, _HW)

    tc_sums = pl.pallas_call(
        _tc_reduce_body,
        grid=(_HW // _NB1,),
        in_specs=[pl.BlockSpec((_B, _CTC, _NB1), lambda h: (0, 0, h))],
        out_specs=pl.BlockSpec((_B, _CTC), lambda h: (0, 0)),
        out_shape=jax.ShapeDtypeStruct((_B, _CTC), jnp.float32),
        scratch_shapes=[pltpu.VMEM((_B, _CTC), jnp.float32)],
    )(x3)
    scp = _sc_partial_sums(x.reshape(_B * _C, _HW))

    dec = pl.pallas_call(
        _combine_body,
        out_shape=jax.ShapeDtypeStruct((1, _B), jnp.int32),
    )(tc_sums, scp, W_ctl, b_ctl.reshape(1, _E)).reshape(_B)

    b3 = b_comp.reshape(_E, _C, 1)
    grid_spec = pltpu.PrefetchScalarGridSpec(
        num_scalar_prefetch=1,
        grid=(_HW // _NB2,),
        in_specs=[
            pl.BlockSpec((_B, _C, _NB2), lambda h, d: (0, 0, h)),
            pl.BlockSpec((1, _C, _C), lambda h, d: (d[0], 0, 0)),
            pl.BlockSpec((1, _C, _C), lambda h, d: (d[1], 0, 0)),
            pl.BlockSpec((1, _C, 1), lambda h, d: (d[0], 0, 0)),
            pl.BlockSpec((1, _C, 1), lambda h, d: (d[1], 0, 0)),
        ],
        out_specs=pl.BlockSpec((_B * _B, _C, _NB2), lambda h, d: (0, 0, h)),
    )
    out = pl.pallas_call(
        _expert_body,
        grid_spec=grid_spec,
        out_shape=jax.ShapeDtypeStruct((_B * _B, _C, _HW), jnp.float32),
    )(dec, x3, W_comp, W_comp, b3, b3)
    return out.reshape(_B * _B, _C, _H, _W)


# speedup vs baseline: 1.2982x; 1.2982x over previous
import jax
import jax.numpy as jnp
from jax.experimental import pallas as pl
from jax.experimental.pallas import tpu as pltpu

_B, _C, _H, _W, _E = 2, 192, 224, 224, 8
_HW = _H * _W
_NB = 3584
_NS = _HW // _NB


def _fused_body(x_ref, wctl_ref, bctl_ref, wcomp_ref, bcomp_ref, o_ref,
                acc_ref, wsel_ref, bsel_ref, dec_ref, sems):
    h = pl.program_id(0)

    @pl.when(h == 0)
    def _():
        acc_ref[...] = jnp.zeros_like(acc_ref)

    @pl.when(h < _NS)
    def _():
        acc_ref[...] += jnp.sum(x_ref[...], axis=2)

    @pl.when(h == _NS - 1)
    def _():
        ctl = jax.lax.dot_general(
            acc_ref[...], wctl_ref[...], (((1,), (1,)), ((), ())),
            preferred_element_type=jnp.float32)
        ctl = ctl * (1.0 / _HW) + bctl_ref[...]
        mx = jnp.max(ctl, axis=1, keepdims=True)
        idx = jax.lax.broadcasted_iota(jnp.int32, (_B, _E), 1)
        dec = jnp.min(jnp.where(ctl == mx, idx, _E), axis=1)
        dec_ref[0] = dec[0]
        dec_ref[1] = dec[1]
        for i in range(_B):
            d = dec_ref[i]
            pltpu.make_async_copy(wcomp_ref.at[d], wsel_ref.at[i],
                                  sems.at[2 * i]).start()
            pltpu.make_async_copy(bcomp_ref.at[d], bsel_ref.at[i],
                                  sems.at[2 * i + 1]).start()

    @pl.when(h == _NS)
    def _():
        for i in range(_B):
            d = dec_ref[i]
            pltpu.make_async_copy(wcomp_ref.at[d], wsel_ref.at[i],
                                  sems.at[2 * i]).wait()
            pltpu.make_async_copy(bcomp_ref.at[d], bsel_ref.at[i],
                                  sems.at[2 * i + 1]).wait()

    @pl.when(h >= _NS)
    def _():
        dims = (((1,), (0,)), ((), ()))
        for i in range(_B):
            w = wsel_ref[i]
            bias = bsel_ref[i]
            for b in range(_B):
                y = jax.lax.dot_general(w, x_ref[b], dims,
                                        preferred_element_type=jnp.float32)
                o_ref[i * _B + b] = y + bias


def kernel(x, W_ctl, b_ctl, W_comp, b_comp):
    x3 = x.reshape(_B, _C, _HW)
    b3 = b_comp.reshape(_E, _C, 1)

    def x_map(h):
        return (0, 0, jnp.where(h < _NS, h, (h - 1) % _NS))

    def o_map(h):
        return (0, 0, jnp.where(h <= _NS, _NS - 1, (h - 1) % _NS))

    out = pl.pallas_call(
        _fused_body,
        grid=(2 * _NS,),
        in_specs=[
            pl.BlockSpec((_B, _C, _NB), x_map),
            pl.BlockSpec((_E, _C), lambda h: (0, 0)),
            pl.BlockSpec((1, _E), lambda h: (0, 0)),
            pl.BlockSpec(memory_space=pltpu.MemorySpace.HBM),
            pl.BlockSpec(memory_space=pltpu.MemorySpace.HBM),
        ],
        out_specs=pl.BlockSpec((_B * _B, _C, _NB), o_map),
        out_shape=jax.ShapeDtypeStruct((_B * _B, _C, _HW), jnp.float32),
        scratch_shapes=[
            pltpu.VMEM((_B, _C), jnp.float32),
            pltpu.VMEM((_B, _C, _C), jnp.float32),
            pltpu.VMEM((_B, _C, 1), jnp.float32),
            pltpu.SMEM((_B,), jnp.int32),
            pltpu.SemaphoreType.DMA((2 * _B,)),
        ],
    )(x3, W_ctl, b_ctl.reshape(1, _E), W_comp, b3)
    return out.reshape(_B * _B, _C, _H, _W)
